# baseline (speedup 1.0000x reference)
import functools

import jax
import jax.numpy as jnp
from jax import lax
from jax.experimental import pallas as pl
from jax.experimental.pallas import tpu as pltpu
from jax.experimental.pallas import tpu_sc as plsc

_B, _S, _D = 4, 2048, 1024
_NC, _NS = 2, 16
_NW = _NC * _NS
_SPW = _S // _NW
_SUB = 16
_NSUB = _SPW // _SUB
_CHUNK = _SUB * _D

_mesh = plsc.VectorSubcoreMesh(core_axis_name="c", subcore_axis_name="s")


@functools.partial(
    pl.kernel,
    mesh=_mesh,
    out_type=jax.ShapeDtypeStruct((_B * _S * _D,), jnp.float32),
    scratch_types=[
        pltpu.VMEM((_CHUNK,), jnp.float32),
        pltpu.VMEM((_CHUNK,), jnp.float32),
    ],
)
def _sc_add(x_hbm, emb_hbm, out_hbm, ebuf, xbuf):
    wid = lax.axis_index("s") * _NC + lax.axis_index("c")
    s0 = wid * _SPW
    for j in range(_NSUB):
        e_off = (s0 + j * _SUB) * _D
        pltpu.sync_copy(emb_hbm.at[pl.ds(e_off, _CHUNK)], ebuf)
        for b in range(_B):
            x_off = b * _S * _D + e_off
            pltpu.sync_copy(x_hbm.at[pl.ds(x_off, _CHUNK)], xbuf)

            def _add(i, _):
                sl = pl.ds(i * 16, 16)
                xbuf[sl] = xbuf[sl] + ebuf[sl]
                return 0

            lax.fori_loop(0, _CHUNK // 16, _add, 0)
            pltpu.sync_copy(xbuf, out_hbm.at[pl.ds(x_off, _CHUNK)])


def kernel(x, emb):
    out = _sc_add(x.reshape(-1), emb.reshape(-1))
    return out.reshape(x.shape)


# speedup vs baseline: 1.6513x; 1.6513x over previous
import functools

import jax
import jax.numpy as jnp
from jax import lax
from jax.experimental import pallas as pl
from jax.experimental.pallas import tpu as pltpu
from jax.experimental.pallas import tpu_sc as plsc

_B, _S, _D = 4, 2048, 1024
_NC, _NS = 2, 16
_NW = _NC * _NS
_SPW = _S // _NW
_SUB = 16
_NSUB = _SPW // _SUB
_CHUNK = _SUB * _D
_STEPS = _NSUB * _B
_NXB = 3

_mesh = plsc.VectorSubcoreMesh(core_axis_name="c", subcore_axis_name="s")


@functools.partial(
    pl.kernel,
    mesh=_mesh,
    out_type=jax.ShapeDtypeStruct((_B * _S * _D,), jnp.float32),
    scratch_types=[
        pltpu.VMEM((_NXB * _CHUNK,), jnp.float32),
        pltpu.VMEM((2 * _CHUNK,), jnp.float32),
        pltpu.SemaphoreType.DMA((_NXB,)),
        pltpu.SemaphoreType.DMA((_NXB,)),
        pltpu.SemaphoreType.DMA((2,)),
    ],
)
def _sc_add(x_hbm, emb_hbm, out_hbm, xbuf, ebuf, xsem, osem, esem):
    wid = lax.axis_index("s") * _NC + lax.axis_index("c")
    s0 = wid * _SPW

    def e_off(j):
        return (s0 + j * _SUB) * _D

    def x_off(t):
        j, b = divmod(t, _B)
        return b * _S * _D + e_off(j)

    def fire_x(t):
        k = t % _NXB
        return pltpu.async_copy(
            x_hbm.at[pl.ds(x_off(t), _CHUNK)],
            xbuf.at[pl.ds(k * _CHUNK, _CHUNK)], xsem.at[k])

    def fire_e(j):
        return pltpu.async_copy(
            emb_hbm.at[pl.ds(e_off(j), _CHUNK)],
            ebuf.at[pl.ds((j % 2) * _CHUNK, _CHUNK)], esem.at[j % 2])

    x_copies = {0: fire_x(0)}
    e_copies = {0: fire_e(0)}
    out_copies = {}

    for t in range(_STEPS):
        j, b = divmod(t, _B)
        k = t % _NXB
        if b == 0:
            e_copies.pop(j).wait()
            if j + 1 < _NSUB:
                e_copies[j + 1] = fire_e(j + 1)
        x_copies.pop(t).wait()
        if t + 1 < _STEPS:
            if t + 1 - _NXB in out_copies:
                out_copies.pop(t + 1 - _NXB).wait()
            x_copies[t + 1] = fire_x(t + 1)

        xb, eb = k * _CHUNK, (j % 2) * _CHUNK

        @plsc.parallel_loop(0, _CHUNK // 16, unroll=8)
        def _add(i):
            plsc.addupdate(
                xbuf.at[pl.ds(xb + i * 16, 16)], ebuf[pl.ds(eb + i * 16, 16)])

        out_copies[t] = pltpu.async_copy(
            xbuf.at[pl.ds(xb, _CHUNK)],
            out_hbm.at[pl.ds(x_off(t), _CHUNK)], osem.at[k])

    for t in sorted(out_copies):
        out_copies.pop(t).wait()


def kernel(x, emb):
    out = _sc_add(x.reshape(-1), emb.reshape(-1))
    return out.reshape(x.shape)


# speedup vs baseline: 4.0418x; 2.4477x over previous
import functools

import jax
import jax.numpy as jnp
from jax import lax
from jax.experimental import pallas as pl
from jax.experimental.pallas import tpu as pltpu
from jax.experimental.pallas import tpu_sc as plsc

_B, _S, _D = 4, 2048, 1024
_NC, _NS = 2, 16
_NW = _NC * _NS
_SPW = _S // _NW
_SUB = 16
_NSUB = _SPW // _SUB
_STEPS = _NSUB * _B
_NXB = 3
_VECS = _SUB * _D // 16

_mesh = plsc.VectorSubcoreMesh(core_axis_name="c", subcore_axis_name="s")


@functools.partial(
    pl.kernel,
    mesh=_mesh,
    out_type=jax.ShapeDtypeStruct((_B, _S, _D), jnp.float32),
    compiler_params=pltpu.CompilerParams(use_tc_tiling_on_sc=True),
    scratch_types=[
        pltpu.VMEM((_NXB, _SUB, _D), jnp.float32),
        pltpu.VMEM((2, _SUB, _D), jnp.float32),
        pltpu.SemaphoreType.DMA((_NXB,)),
        pltpu.SemaphoreType.DMA((_NXB,)),
        pltpu.SemaphoreType.DMA((2,)),
    ],
)
def _sc_add(x_hbm, emb_hbm, out_hbm, xbuf, ebuf, xsem, osem, esem):
    wid = lax.axis_index("s") * _NC + lax.axis_index("c")
    s0 = wid * _SPW

    def rows(j):
        return pl.ds(s0 + j * _SUB, _SUB)

    def fire_x(t):
        j, b = divmod(t, _B)
        k = t % _NXB
        return pltpu.async_copy(x_hbm.at[b, rows(j)], xbuf.at[k], xsem.at[k])

    def fire_e(j):
        return pltpu.async_copy(emb_hbm.at[rows(j)], ebuf.at[j % 2], esem.at[j % 2])

    x_copies = {0: fire_x(0)}
    e_copies = {0: fire_e(0)}
    out_copies = {}

    for t in range(_STEPS):
        j, b = divmod(t, _B)
        k = t % _NXB
        if b == 0:
            e_copies.pop(j).wait()
            if j + 1 < _NSUB:
                e_copies[j + 1] = fire_e(j + 1)
        x_copies.pop(t).wait()
        if t + 1 < _STEPS:
            if t + 1 - _NXB in out_copies:
                out_copies.pop(t + 1 - _NXB).wait()
            x_copies[t + 1] = fire_x(t + 1)

        je = j % 2

        @plsc.parallel_loop(0, _VECS, unroll=8)
        def _add(i):
            r = i >> 6
            c = (i & 63) * 16
            plsc.addupdate(xbuf.at[k, r, pl.ds(c, 16)], ebuf[je, r, pl.ds(c, 16)])

        out_copies[t] = pltpu.async_copy(
            xbuf.at[k], out_hbm.at[b, rows(j)], osem.at[k])

    for t in sorted(out_copies):
        out_copies.pop(t).wait()


def kernel(x, emb):
    return _sc_add(x, emb)


# speedup vs baseline: 4.3276x; 1.0707x over previous
import functools

import jax
import jax.numpy as jnp
from jax import lax
from jax.experimental import pallas as pl
from jax.experimental.pallas import tpu as pltpu
from jax.experimental.pallas import tpu_sc as plsc

_B, _S, _D = 4, 2048, 1024
_NC, _NS = 2, 16
_NW = _NC * _NS
_SPW = _S // _NW
_SUB = 16
_NSUB = _SPW // _SUB
_STEPS = _NSUB * _B
_NXB = 5
_AHEAD = 3
_VECS = _SUB * _D // 16

_mesh = plsc.VectorSubcoreMesh(core_axis_name="c", subcore_axis_name="s")


@functools.partial(
    pl.kernel,
    mesh=_mesh,
    out_type=jax.ShapeDtypeStruct((_B, _S, _D), jnp.float32),
    compiler_params=pltpu.CompilerParams(use_tc_tiling_on_sc=True),
    scratch_types=[
        pltpu.VMEM((_NXB, _SUB, _D), jnp.float32),
        pltpu.VMEM((2, _SUB, _D), jnp.float32),
        pltpu.SemaphoreType.DMA((_NXB,)),
        pltpu.SemaphoreType.DMA((_NXB,)),
        pltpu.SemaphoreType.DMA((2,)),
    ],
)
def _sc_add(x_hbm, emb_hbm, out_hbm, xbuf, ebuf, xsem, osem, esem):
    wid = lax.axis_index("s") * _NC + lax.axis_index("c")
    s0 = wid * _SPW

    def rows(j):
        return pl.ds(s0 + j * _SUB, _SUB)

    def fire_x(t):
        j, b = divmod(t, _B)
        k = t % _NXB
        return pltpu.async_copy(x_hbm.at[b, rows(j)], xbuf.at[k], xsem.at[k])

    def fire_e(j):
        return pltpu.async_copy(emb_hbm.at[rows(j)], ebuf.at[j % 2], esem.at[j % 2])

    x_copies = {t: fire_x(t) for t in range(_AHEAD)}
    e_copies = {0: fire_e(0), 1: fire_e(1)}
    out_copies = {}

    for t in range(_STEPS):
        j, b = divmod(t, _B)
        k = t % _NXB
        if b == 0:
            e_copies.pop(j).wait()
        x_copies.pop(t).wait()
        t3 = t + _AHEAD
        if t3 < _STEPS:
            if t3 - _NXB in out_copies:
                out_copies.pop(t3 - _NXB).wait()
            x_copies[t3] = fire_x(t3)

        je = j % 2

        @plsc.parallel_loop(0, _VECS, unroll=8)
        def _add(i):
            r = i >> 6
            c = (i & 63) * 16
            plsc.addupdate(xbuf.at[k, r, pl.ds(c, 16)], ebuf[je, r, pl.ds(c, 16)])

        out_copies[t] = pltpu.async_copy(
            xbuf.at[k], out_hbm.at[b, rows(j)], osem.at[k])
        if b == _B - 1 and j + 2 < _NSUB:
            e_copies[j + 2] = fire_e(j + 2)

    for t in sorted(out_copies):
        out_copies.pop(t).wait()


def kernel(x, emb):
    return _sc_add(x, emb)
